# baseline (speedup 1.0000x reference)
import functools

import jax
import jax.numpy as jnp
from jax import lax
from jax.experimental import pallas as pl
from jax.experimental.pallas import tpu as pltpu
from jax.experimental.pallas import tpu_sc as plsc

NUM_TIMES = 100000
NUM_USERS = 1000000
EMBED_DIM = 32
BATCH = 16384
ITEM_LO = NUM_TIMES + NUM_USERS
TOTAL_ROWS = NUM_TIMES + NUM_USERS + 1000000
PACK = 128 // EMBED_DIM
PACKED_ROWS = TOTAL_ROWS // PACK

NC = 1
NS = 16
L = 16
NW = NC * NS
BPW = BATCH // NW
CHUNK = 128
NCHUNK = BPW // CHUNK
NGROUP = CHUNK // L

_GDN = lax.GatherDimensionNumbers(
    offset_dims=(), collapsed_slice_dims=(0,), start_index_map=(0,))


def _lane_perm(v, perm):
    return lax.gather(v, perm[:, None], dimension_numbers=_GDN,
                      slice_sizes=(1,), mode=lax.GatherScatterMode.PROMISE_IN_BOUNDS)


def _body(tcol_hbm, icol_hbm, tab_hbm, out_hbm,
          idx_t, idx_u, idx_i, off_t, off_u, off_i,
          rows_t, rows_u, rows_i, outv, sem0, sem1):
    wid = lax.axis_index("s") * NC + lax.axis_index("c")
    base = wid * BPW

    for j in range(NCHUNK):
        pltpu.sync_copy(tcol_hbm.at[pl.ds(base + j * CHUNK, CHUNK)], idx_t.at[j])
        pltpu.sync_copy(icol_hbm.at[pl.ds(base + j * CHUNK, CHUNK)], idx_i.at[j])

    for j in range(NCHUNK):
        for o in range(0, CHUNK, L):
            t = idx_t[j, pl.ds(o, L)]
            u = t + NUM_TIMES
            i = idx_i[j, pl.ds(o, L)] + ITEM_LO
            off_t[j, pl.ds(o, L)] = (t & (PACK - 1)) * EMBED_DIM
            off_u[j, pl.ds(o, L)] = (u & (PACK - 1)) * EMBED_DIM
            off_i[j, pl.ds(o, L)] = (i & (PACK - 1)) * EMBED_DIM
            idx_t[j, pl.ds(o, L)] = t >> 2
            idx_u[j, pl.ds(o, L)] = u >> 2
            idx_i[j, pl.ds(o, L)] = i >> 2

    sems = [sem0, sem1]

    def fire(j):
        s = j % 2
        return [
            pltpu.async_copy(tab_hbm.at[idx_t.at[j]], rows_t.at[s], sems[s]),
            pltpu.async_copy(tab_hbm.at[idx_u.at[j]], rows_u.at[s], sems[s]),
            pltpu.async_copy(tab_hbm.at[idx_i.at[j]], rows_i.at[s], sems[s]),
        ]

    iota = lax.broadcasted_iota(jnp.int32, (L,), 0)
    perms = [iota ^ s for s in (8, 4, 2, 1)]
    masks = [iota == r for r in range(L)]

    def compute(j):
        s = j % 2

        def gbody(g, _):
            r0 = g * L
            ot = off_t[j, pl.ds(r0, L)]
            ou = off_u[j, pl.ds(r0, L)]
            oi = off_i[j, pl.ds(r0, L)]
            acc = jnp.zeros((L,), jnp.float32)
            for r in range(L):
                rr = r0 + r
                qt = ot[r]
                qu = ou[r]
                qi = oi[r]
                q = (rows_t[s, rr, pl.ds(qt, L)]
                     * rows_u[s, rr, pl.ds(qu, L)]
                     * rows_i[s, rr, pl.ds(qi, L)]
                     + rows_t[s, rr, pl.ds(qt + L, L)]
                     * rows_u[s, rr, pl.ds(qu + L, L)]
                     * rows_i[s, rr, pl.ds(qi + L, L)])
                for p in perms:
                    q = q + _lane_perm(q, p)
                acc = jnp.where(masks[r], q, acc)
            outv[pl.ds(j * CHUNK + r0, L)] = acc
            return 0

        lax.fori_loop(0, NGROUP, gbody, 0)

    inflight = fire(0)
    for j in range(NCHUNK):
        nxt = fire(j + 1) if j + 1 < NCHUNK else []
        for c in inflight:
            c.wait()
        inflight = nxt
        compute(j)

    pltpu.sync_copy(outv, out_hbm.at[pl.ds(base, BPW)])


def kernel(x, embedding):
    x = x.astype(jnp.int32)
    tcol = x[:, 0]
    icol = x[:, 2]
    tab = embedding.reshape(PACKED_ROWS, 128)
    run = functools.partial(
        pl.kernel,
        mesh=plsc.VectorSubcoreMesh(core_axis_name="c", subcore_axis_name="s", num_cores=1),
        compiler_params=pltpu.CompilerParams(use_tc_tiling_on_sc=True),
        out_type=jax.ShapeDtypeStruct((BATCH,), jnp.float32),
        scratch_types=[
            pltpu.VMEM((NCHUNK, CHUNK), jnp.int32),
            pltpu.VMEM((NCHUNK, CHUNK), jnp.int32),
            pltpu.VMEM((NCHUNK, CHUNK), jnp.int32),
            pltpu.VMEM((NCHUNK, CHUNK), jnp.int32),
            pltpu.VMEM((NCHUNK, CHUNK), jnp.int32),
            pltpu.VMEM((NCHUNK, CHUNK), jnp.int32),
            pltpu.VMEM((2, CHUNK, 128), jnp.float32),
            pltpu.VMEM((2, CHUNK, 128), jnp.float32),
            pltpu.VMEM((2, CHUNK, 128), jnp.float32),
            pltpu.VMEM((BPW,), jnp.float32),
            pltpu.SemaphoreType.DMA,
            pltpu.SemaphoreType.DMA,
        ],
    )(_body)
    return run(tcol, icol, tab)


# speedup vs baseline: 5.3194x; 5.3194x over previous
import functools

import jax
import jax.numpy as jnp
from jax import lax
from jax.experimental import pallas as pl
from jax.experimental.pallas import tpu as pltpu
from jax.experimental.pallas import tpu_sc as plsc

NUM_TIMES = 100000
NUM_USERS = 1000000
EMBED_DIM = 32
BATCH = 16384
ITEM_LO = NUM_TIMES + NUM_USERS
PACK = 128 // EMBED_DIM
PTU = 2 * NUM_TIMES // PACK
PUOFF = NUM_TIMES // PACK

NC = 1
NS = 16
L = 16
NW = NC * NS
BPW = BATCH // NW
CHUNK = 128
NCHUNK = BPW // CHUNK
NGROUP = CHUNK // L

_GDN = lax.GatherDimensionNumbers(
    offset_dims=(), collapsed_slice_dims=(0,), start_index_map=(0,))


def _lane_perm(v, perm):
    return lax.gather(v, perm[:, None], dimension_numbers=_GDN,
                      slice_sizes=(1,), mode=lax.GatherScatterMode.PROMISE_IN_BOUNDS)


def _body(tcol_hbm, icol_hbm, tab_tu_hbm, tab_i_hbm, out_hbm,
          idx_t, idx_u, idx_i, off_t, off_i,
          rows_t, rows_u, rows_i, outv, sem0, sem1):
    wid = lax.axis_index("s") * NC + lax.axis_index("c")
    base = wid * BPW

    for j in range(NCHUNK):
        pltpu.sync_copy(tcol_hbm.at[pl.ds(base + j * CHUNK, CHUNK)], idx_t.at[j])
        pltpu.sync_copy(icol_hbm.at[pl.ds(base + j * CHUNK, CHUNK)], idx_i.at[j])

    for j in range(NCHUNK):
        for o in range(0, CHUNK, L):
            t = idx_t[j, pl.ds(o, L)]
            i = idx_i[j, pl.ds(o, L)]
            off_t[j, pl.ds(o, L)] = (t & (PACK - 1)) * EMBED_DIM
            off_i[j, pl.ds(o, L)] = (i & (PACK - 1)) * EMBED_DIM
            pt = t >> 2
            idx_t[j, pl.ds(o, L)] = pt
            idx_u[j, pl.ds(o, L)] = pt + PUOFF
            idx_i[j, pl.ds(o, L)] = i >> 2

    sems = [sem0, sem1]

    def fire(j):
        s = j % 2
        return [
            pltpu.async_copy(tab_tu_hbm.at[idx_t.at[j]], rows_t.at[s], sems[s]),
            pltpu.async_copy(tab_tu_hbm.at[idx_u.at[j]], rows_u.at[s], sems[s]),
            pltpu.async_copy(tab_i_hbm.at[idx_i.at[j]], rows_i.at[s], sems[s]),
        ]

    iota = lax.broadcasted_iota(jnp.int32, (L,), 0)
    perms = [iota ^ s for s in (8, 4, 2, 1)]
    masks = [iota == r for r in range(L)]

    def compute(j):
        s = j % 2

        def gbody(g, _):
            r0 = g * L
            ot = off_t[j, pl.ds(r0, L)]
            oi = off_i[j, pl.ds(r0, L)]
            acc = jnp.zeros((L,), jnp.float32)
            for r in range(L):
                rr = r0 + r
                qt = ot[r]
                qi = oi[r]
                q = (rows_t[s, rr, pl.ds(qt, L)]
                     * rows_u[s, rr, pl.ds(qt, L)]
                     * rows_i[s, rr, pl.ds(qi, L)]
                     + rows_t[s, rr, pl.ds(qt + L, L)]
                     * rows_u[s, rr, pl.ds(qt + L, L)]
                     * rows_i[s, rr, pl.ds(qi + L, L)])
                for p in perms:
                    q = q + _lane_perm(q, p)
                acc = jnp.where(masks[r], q, acc)
            outv[pl.ds(j * CHUNK + r0, L)] = acc
            return 0

        lax.fori_loop(0, NGROUP, gbody, 0)

    inflight = fire(0)
    for j in range(NCHUNK):
        nxt = fire(j + 1) if j + 1 < NCHUNK else []
        for c in inflight:
            c.wait()
        inflight = nxt
        compute(j)

    pltpu.sync_copy(outv, out_hbm.at[pl.ds(base, BPW)])


def kernel(x, embedding):
    x = x.astype(jnp.int32)
    tcol = x[:, 0]
    icol = x[:, 2]
    tab_tu = lax.slice(embedding, (0, 0), (2 * NUM_TIMES, EMBED_DIM)).reshape(PTU, 128)
    tab_i = lax.slice(embedding, (ITEM_LO, 0),
                      (ITEM_LO + NUM_TIMES, EMBED_DIM)).reshape(PTU // 2, 128)
    run = functools.partial(
        pl.kernel,
        mesh=plsc.VectorSubcoreMesh(core_axis_name="c", subcore_axis_name="s", num_cores=1),
        compiler_params=pltpu.CompilerParams(use_tc_tiling_on_sc=True),
        out_type=jax.ShapeDtypeStruct((BATCH,), jnp.float32),
        scratch_types=[
            pltpu.VMEM((NCHUNK, CHUNK), jnp.int32),
            pltpu.VMEM((NCHUNK, CHUNK), jnp.int32),
            pltpu.VMEM((NCHUNK, CHUNK), jnp.int32),
            pltpu.VMEM((NCHUNK, CHUNK), jnp.int32),
            pltpu.VMEM((NCHUNK, CHUNK), jnp.int32),
            pltpu.VMEM((2, CHUNK, 128), jnp.float32),
            pltpu.VMEM((2, CHUNK, 128), jnp.float32),
            pltpu.VMEM((2, CHUNK, 128), jnp.float32),
            pltpu.VMEM((BPW,), jnp.float32),
            pltpu.SemaphoreType.DMA,
            pltpu.SemaphoreType.DMA,
        ],
    )(_body)
    return run(tcol, icol, tab_tu, tab_i)


# speedup vs baseline: 7.6902x; 1.4457x over previous
import functools

import jax
import jax.numpy as jnp
from jax import lax
from jax.experimental import pallas as pl
from jax.experimental.pallas import tpu as pltpu
from jax.experimental.pallas import tpu_sc as plsc

NUM_TIMES = 100000
NUM_USERS = 1000000
EMBED_DIM = 32
BATCH = 16384
TOTAL_ROWS = 2100000
ITEM_LO = NUM_TIMES + NUM_USERS
PACK = 128 // EMBED_DIM

CBLK = 4096
OBLK = CBLK // PACK
NTU_BLK = (2 * NUM_TIMES + CBLK - 1) // CBLK
ITEM_BLK0 = ITEM_LO // CBLK
ITEM_SKEW = ITEM_LO - ITEM_BLK0 * CBLK
NI_BLK = (ITEM_SKEW + NUM_TIMES + CBLK - 1) // CBLK
NBLK = NTU_BLK + NI_BLK
PACKED_OUT = NBLK * OBLK
PI_BASE = NTU_BLK * OBLK

NC = 1
NS = 16
L = 16
NW = NC * NS
BPW = BATCH // NW
CHUNK = 128
NCHUNK = BPW // CHUNK
NGROUP = CHUNK // L

_GDN = lax.GatherDimensionNumbers(
    offset_dims=(), collapsed_slice_dims=(0,), start_index_map=(0,))


def _lane_perm(v, perm):
    return lax.gather(v, perm[:, None], dimension_numbers=_GDN,
                      slice_sizes=(1,), mode=lax.GatherScatterMode.PROMISE_IN_BOUNDS)


def _transpose_body(v_ref, out_ref):
    x = v_ref[...]
    parts = [x[:, OBLK * q:OBLK * (q + 1)].T for q in range(PACK)]
    out_ref[...] = jnp.concatenate(parts, axis=1)


def _pack_windows(vt):
    return pl.pallas_call(
        _transpose_body,
        grid=(NBLK,),
        in_specs=[pl.BlockSpec((EMBED_DIM, CBLK),
                               lambda k: (0, jnp.where(k < NTU_BLK, k, k + (ITEM_BLK0 - NTU_BLK))))],
        out_specs=pl.BlockSpec((OBLK, 128), lambda k: (k, 0)),
        out_shape=jax.ShapeDtypeStruct((PACKED_OUT, 128), jnp.float32),
    )(vt)


def _prow(e):
    return ((e >> 12) << 10) + (e & (OBLK - 1))


def _poff(e):
    return ((e >> 10) & (PACK - 1)) * EMBED_DIM


def _body(tcol_hbm, icol_hbm, tab_hbm, out_hbm,
          idx_t, idx_u, idx_i, off_t, off_u, off_i,
          rows_t, rows_u, rows_i, outv, sem0, sem1):
    wid = lax.axis_index("s") * NC + lax.axis_index("c")
    base = wid * BPW

    for j in range(NCHUNK):
        pltpu.sync_copy(tcol_hbm.at[pl.ds(base + j * CHUNK, CHUNK)], idx_t.at[j])
        pltpu.sync_copy(icol_hbm.at[pl.ds(base + j * CHUNK, CHUNK)], idx_i.at[j])

    for j in range(NCHUNK):
        for o in range(0, CHUNK, L):
            t = idx_t[j, pl.ds(o, L)]
            u = t + NUM_TIMES
            ec = idx_i[j, pl.ds(o, L)] + ITEM_SKEW
            off_t[j, pl.ds(o, L)] = _poff(t)
            off_u[j, pl.ds(o, L)] = _poff(u)
            off_i[j, pl.ds(o, L)] = _poff(ec)
            idx_t[j, pl.ds(o, L)] = _prow(t)
            idx_u[j, pl.ds(o, L)] = _prow(u)
            idx_i[j, pl.ds(o, L)] = _prow(ec) + PI_BASE

    sems = [sem0, sem1]

    def fire(j):
        s = j % 2
        return [
            pltpu.async_copy(tab_hbm.at[idx_t.at[j]], rows_t.at[s], sems[s]),
            pltpu.async_copy(tab_hbm.at[idx_u.at[j]], rows_u.at[s], sems[s]),
            pltpu.async_copy(tab_hbm.at[idx_i.at[j]], rows_i.at[s], sems[s]),
        ]

    iota = lax.broadcasted_iota(jnp.int32, (L,), 0)
    perms = [iota ^ s for s in (8, 4, 2, 1)]
    masks = [iota == r for r in range(L)]

    def compute(j):
        s = j % 2

        def gbody(g, _):
            r0 = g * L
            ot = off_t[j, pl.ds(r0, L)]
            ou = off_u[j, pl.ds(r0, L)]
            oi = off_i[j, pl.ds(r0, L)]
            acc = jnp.zeros((L,), jnp.float32)
            for r in range(L):
                rr = r0 + r
                qt = ot[r]
                qu = ou[r]
                qi = oi[r]
                q = (rows_t[s, rr, pl.ds(qt, L)]
                     * rows_u[s, rr, pl.ds(qu, L)]
                     * rows_i[s, rr, pl.ds(qi, L)]
                     + rows_t[s, rr, pl.ds(qt + L, L)]
                     * rows_u[s, rr, pl.ds(qu + L, L)]
                     * rows_i[s, rr, pl.ds(qi + L, L)])
                for p in perms:
                    q = q + _lane_perm(q, p)
                acc = jnp.where(masks[r], q, acc)
            outv[pl.ds(j * CHUNK + r0, L)] = acc
            return 0

        lax.fori_loop(0, NGROUP, gbody, 0)

    inflight = fire(0)
    for j in range(NCHUNK):
        nxt = fire(j + 1) if j + 1 < NCHUNK else []
        for c in inflight:
            c.wait()
        inflight = nxt
        compute(j)

    pltpu.sync_copy(outv, out_hbm.at[pl.ds(base, BPW)])


def kernel(x, embedding):
    x = x.astype(jnp.int32)
    tcol = x[:, 0]
    icol = x[:, 2]
    vt = jnp.swapaxes(embedding, 0, 1)
    tab = _pack_windows(vt)
    run = functools.partial(
        pl.kernel,
        mesh=plsc.VectorSubcoreMesh(core_axis_name="c", subcore_axis_name="s", num_cores=1),
        compiler_params=pltpu.CompilerParams(use_tc_tiling_on_sc=True),
        out_type=jax.ShapeDtypeStruct((BATCH,), jnp.float32),
        scratch_types=[
            pltpu.VMEM((NCHUNK, CHUNK), jnp.int32),
            pltpu.VMEM((NCHUNK, CHUNK), jnp.int32),
            pltpu.VMEM((NCHUNK, CHUNK), jnp.int32),
            pltpu.VMEM((NCHUNK, CHUNK), jnp.int32),
            pltpu.VMEM((NCHUNK, CHUNK), jnp.int32),
            pltpu.VMEM((NCHUNK, CHUNK), jnp.int32),
            pltpu.VMEM((2, CHUNK, 128), jnp.float32),
            pltpu.VMEM((2, CHUNK, 128), jnp.float32),
            pltpu.VMEM((2, CHUNK, 128), jnp.float32),
            pltpu.VMEM((BPW,), jnp.float32),
            pltpu.SemaphoreType.DMA,
            pltpu.SemaphoreType.DMA,
        ],
    )(_body)
    return run(tcol, icol, tab)
